# baseline (speedup 1.0000x reference)
import functools

import jax
import jax.numpy as jnp
from jax import lax
from jax.experimental import pallas as pl
from jax.experimental.pallas import tpu as pltpu
from jax.experimental.pallas import tpu_sc as plsc

NC = 2
NS = 16
NW = NC * NS

TB = 128
NBUF = 4
SB = 4
TPV = 32768


def _table_prep(tableT):
    D, V = tableT.shape
    nblk = pl.cdiv(V, TPV)
    H = TPV // 2

    def body(x_ref, y_ref):
        xT = x_ref[...].T
        y_ref[...] = jnp.concatenate([xT[:H], xT[H:]], axis=1)

    return pl.pallas_call(
        body,
        grid=(nblk,),
        in_specs=[pl.BlockSpec((D, TPV), lambda i: (0, i))],
        out_specs=pl.BlockSpec((H, 2 * D), lambda i: (i, 0)),
        out_shape=jax.ShapeDtypeStruct((nblk * H, 2 * D), jnp.float32),
    )(tableT)


def _remap(idx):
    blk = idx // TPV
    j = idx % TPV
    return blk * TPV + 2 * (j % (TPV // 2)) + j // (TPV // 2)


def _out_transpose(x, S, BB, D, Ssl, k, acc=None):
    C8 = D // 8

    def compute(x_ref, y_ref):
        for s2 in range(SB):
            xs = x_ref[s2 * BB : (s2 + 1) * BB]
            ya = jnp.transpose(xs[:, :, :D], (0, 2, 1))
            yo = jnp.transpose(xs[:, :, D:], (0, 2, 1))
            y = jnp.concatenate([ya, yo], axis=2)
            y_ref[s2] = y.reshape(BB, C8, 8, TB).transpose(1, 0, 2, 3)

    def body(x_ref, y_ref, *_):
        compute(x_ref, y_ref)

    in_specs = [pl.BlockSpec((SB * BB, TB // 2, 2 * D), lambda s: (s, 0, 0))]
    args = [x]
    kwargs = {}
    if acc is not None:
        def body(x_ref, acc_ref, y_ref):
            compute(x_ref, y_ref)

        in_specs = in_specs + [pl.BlockSpec(memory_space=pl.ANY)]
        args = [x, acc]
        kwargs = {"input_output_aliases": {1: 0}}

    return pl.pallas_call(
        body,
        grid=(Ssl // SB,),
        in_specs=in_specs,
        out_specs=pl.BlockSpec(
            (SB, C8, BB, 8, TB),
            lambda s: (s + k * (Ssl // SB), 0, 0, 0, 0),
        ),
        out_shape=jax.ShapeDtypeStruct((S, C8, BB, 8, TB), jnp.float32),
        **kwargs,
    )(*args)


def _make_sc_gather(n_units, V2, D):
    upw = n_units // NW
    mesh = plsc.VectorSubcoreMesh(core_axis_name="c", subcore_axis_name="s")
    N = n_units * TB

    @functools.partial(
        pl.kernel,
        mesh=mesh,
        out_type=jax.ShapeDtypeStruct((N, D), jnp.float32),
        scratch_types=[
            pltpu.VMEM((upw, TB), jnp.int32),
            pltpu.VMEM((NBUF, TB, D), jnp.float32),
        ]
        + [pltpu.SemaphoreType.DMA] * (2 * NBUF),
        compiler_params=pltpu.CompilerParams(use_tc_tiling_on_sc=False),
    )
    def sc_gather(idx_hbm, table_hbm, out_hbm, idx_v, rows_v, *sems):
        gsems = sems[:NBUF]
        wsems = sems[NBUF:]
        wid = lax.axis_index("s") * NC + lax.axis_index("c")
        ubase = wid * upw

        pltpu.sync_copy(idx_hbm.at[pl.ds(ubase, upw)], idx_v)

        def gather_start(ul, b):
            pltpu.make_async_copy(
                table_hbm.at[idx_v.at[ul]], rows_v.at[b], gsems[b]
            ).start()

        def gather_wait(b):
            pltpu.make_async_copy(
                table_hbm.at[idx_v.at[0]], rows_v.at[b], gsems[b]
            ).wait()

        def write_start(ul, b):
            pltpu.make_async_copy(
                rows_v.at[b],
                out_hbm.at[pl.ds((ubase + ul) * TB, TB)],
                wsems[b],
            ).start()

        def write_wait(b):
            pltpu.make_async_copy(
                rows_v.at[b], out_hbm.at[pl.ds(0, TB)], wsems[b]
            ).wait()

        for b in range(NBUF):
            gather_start(b, b)

        def outer(g, carry):
            for b in range(NBUF):
                ul = g * NBUF + b
                gather_wait(b)
                write_start(ul, b)
            for b in range(NBUF):
                jn = (g + 1) * NBUF + b

                @pl.when(jn < upw)
                def _():
                    write_wait(b)
                    gather_start(jn, b)

            return carry

        lax.fori_loop(0, upw // NBUF, outer, 0)

        for b in range(NBUF):
            write_wait(b)

    return sc_gather


SLABS = 10


def kernel(sequence, table):
    B, S = sequence.shape
    V, D = table.shape
    BB = B // TB
    n_units = S * BB
    assert n_units % NW == 0 and D % 16 == 0

    table2 = _table_prep(table.T)
    V2 = 2 * table2.shape[0]
    table2v = table2.reshape(V2, D)

    idx = (
        _remap(sequence.T)
        .reshape(n_units, 2, TB // 2)
        .transpose(0, 2, 1)
        .reshape(n_units, TB)
    )
    Ssl = S // SLABS
    usl = Ssl * BB
    sc = _make_sc_gather(usl, V2, D)
    out5d = None
    for k in range(SLABS):
        rows_k = sc(idx[k * usl : (k + 1) * usl], table2v)
        out5d = _out_transpose(
            rows_k.reshape(usl, TB // 2, 2 * D), S, BB, D, Ssl, k, out5d
        )

    return out5d.transpose(2, 4, 0, 1, 3).reshape(B, S, D)


# speedup vs baseline: 1.0261x; 1.0261x over previous
import functools

import jax
import jax.numpy as jnp
from jax import lax
from jax.experimental import pallas as pl
from jax.experimental.pallas import tpu as pltpu
from jax.experimental.pallas import tpu_sc as plsc

NC = 2
NS = 16
NW = NC * NS

TB = 128
NBUF = 4
SB = 8
TPV = 32768


def _table_prep(tableT):
    D, V = tableT.shape
    nblk = pl.cdiv(V, TPV)
    H = TPV // 2

    def body(x_ref, y_ref):
        xT = x_ref[...].T
        y_ref[...] = jnp.concatenate([xT[:H], xT[H:]], axis=1)

    return pl.pallas_call(
        body,
        grid=(nblk,),
        in_specs=[pl.BlockSpec((D, TPV), lambda i: (0, i))],
        out_specs=pl.BlockSpec((H, 2 * D), lambda i: (i, 0)),
        out_shape=jax.ShapeDtypeStruct((nblk * H, 2 * D), jnp.float32),
    )(tableT)


def _remap(idx):
    blk = idx // TPV
    j = idx % TPV
    return blk * TPV + 2 * (j % (TPV // 2)) + j // (TPV // 2)


def _out_transpose(x, S, BB, D, Ssl, k, acc=None):
    C8 = D // 8

    def compute(x_ref, y_ref):
        for s2 in range(SB):
            xs = x_ref[s2 * BB : (s2 + 1) * BB]
            ya = jnp.transpose(xs[:, :, :D], (0, 2, 1))
            yo = jnp.transpose(xs[:, :, D:], (0, 2, 1))
            y = jnp.concatenate([ya, yo], axis=2)
            y_ref[s2] = y.reshape(BB, C8, 8, TB).transpose(1, 0, 2, 3)

    def body(x_ref, y_ref, *_):
        compute(x_ref, y_ref)

    in_specs = [pl.BlockSpec((SB * BB, TB // 2, 2 * D), lambda s: (s, 0, 0))]
    args = [x]
    kwargs = {}
    if acc is not None:
        def body(x_ref, acc_ref, y_ref):
            compute(x_ref, y_ref)

        in_specs = in_specs + [pl.BlockSpec(memory_space=pl.ANY)]
        args = [x, acc]
        kwargs = {"input_output_aliases": {1: 0}}

    return pl.pallas_call(
        body,
        grid=(Ssl // SB,),
        in_specs=in_specs,
        out_specs=pl.BlockSpec(
            (SB, C8, BB, 8, TB),
            lambda s: (s + k * (Ssl // SB), 0, 0, 0, 0),
        ),
        out_shape=jax.ShapeDtypeStruct((S, C8, BB, 8, TB), jnp.float32),
        **kwargs,
    )(*args)


def _make_sc_gather(n_units, V2, D):
    upw = n_units // NW
    mesh = plsc.VectorSubcoreMesh(core_axis_name="c", subcore_axis_name="s")
    N = n_units * TB

    @functools.partial(
        pl.kernel,
        mesh=mesh,
        out_type=jax.ShapeDtypeStruct((N, D), jnp.float32),
        scratch_types=[
            pltpu.VMEM((upw, TB), jnp.int32),
            pltpu.VMEM((NBUF, TB, D), jnp.float32),
        ]
        + [pltpu.SemaphoreType.DMA] * (2 * NBUF),
        compiler_params=pltpu.CompilerParams(use_tc_tiling_on_sc=False),
    )
    def sc_gather(idx_hbm, table_hbm, out_hbm, idx_v, rows_v, *sems):
        gsems = sems[:NBUF]
        wsems = sems[NBUF:]
        wid = lax.axis_index("s") * NC + lax.axis_index("c")
        ubase = wid * upw

        pltpu.sync_copy(idx_hbm.at[pl.ds(ubase, upw)], idx_v)

        def gather_start(ul, b):
            pltpu.make_async_copy(
                table_hbm.at[idx_v.at[ul]], rows_v.at[b], gsems[b]
            ).start()

        def gather_wait(b):
            pltpu.make_async_copy(
                table_hbm.at[idx_v.at[0]], rows_v.at[b], gsems[b]
            ).wait()

        def write_start(ul, b):
            pltpu.make_async_copy(
                rows_v.at[b],
                out_hbm.at[pl.ds((ubase + ul) * TB, TB)],
                wsems[b],
            ).start()

        def write_wait(b):
            pltpu.make_async_copy(
                rows_v.at[b], out_hbm.at[pl.ds(0, TB)], wsems[b]
            ).wait()

        for b in range(NBUF):
            gather_start(b, b)

        def outer(g, carry):
            for b in range(NBUF):
                ul = g * NBUF + b
                gather_wait(b)
                write_start(ul, b)
            for b in range(NBUF):
                jn = (g + 1) * NBUF + b

                @pl.when(jn < upw)
                def _():
                    write_wait(b)
                    gather_start(jn, b)

            return carry

        lax.fori_loop(0, upw // NBUF, outer, 0)

        for b in range(NBUF):
            write_wait(b)

    return sc_gather


SLABS = 5


def kernel(sequence, table):
    B, S = sequence.shape
    V, D = table.shape
    BB = B // TB
    n_units = S * BB
    assert n_units % NW == 0 and D % 16 == 0

    table2 = _table_prep(table.T)
    V2 = 2 * table2.shape[0]
    table2v = table2.reshape(V2, D)

    idx = (
        _remap(sequence.T)
        .reshape(n_units, 2, TB // 2)
        .transpose(0, 2, 1)
        .reshape(n_units, TB)
    )
    Ssl = S // SLABS
    usl = Ssl * BB
    sc = _make_sc_gather(usl, V2, D)
    out5d = None
    for k in range(SLABS):
        rows_k = sc(idx[k * usl : (k + 1) * usl], table2v)
        out5d = _out_transpose(
            rows_k.reshape(usl, TB // 2, 2 * D), S, BB, D, Ssl, k, out5d
        )

    return out5d.transpose(2, 4, 0, 1, 3).reshape(B, S, D)


# speedup vs baseline: 1.0413x; 1.0148x over previous
import functools

import jax
import jax.numpy as jnp
from jax import lax
from jax.experimental import pallas as pl
from jax.experimental.pallas import tpu as pltpu
from jax.experimental.pallas import tpu_sc as plsc

NC = 2
NS = 16
NW = NC * NS

TB = 128
NBUF = 4
SB = 4
TPV = 32768


def _table_prep(tableT):
    D, V = tableT.shape
    nblk = pl.cdiv(V, TPV)
    H = TPV // 2

    def body(x_ref, y_ref):
        xT = x_ref[...].T
        y_ref[...] = jnp.concatenate([xT[:H], xT[H:]], axis=1)

    return pl.pallas_call(
        body,
        grid=(nblk,),
        in_specs=[pl.BlockSpec((D, TPV), lambda i: (0, i))],
        out_specs=pl.BlockSpec((H, 2 * D), lambda i: (i, 0)),
        out_shape=jax.ShapeDtypeStruct((nblk * H, 2 * D), jnp.float32),
    )(tableT)


def _remap(idx):
    blk = idx // TPV
    j = idx % TPV
    return blk * TPV + 2 * (j % (TPV // 2)) + j // (TPV // 2)


def _out_transpose(x, S, BB, D, Ssl, k, acc=None):
    C8 = D // 8

    def compute(x_ref, y_ref):
        for s2 in range(SB):
            xs = x_ref[s2 * BB : (s2 + 1) * BB]
            ya = jnp.transpose(xs[:, :, :D], (0, 2, 1))
            yo = jnp.transpose(xs[:, :, D:], (0, 2, 1))
            y = jnp.concatenate([ya, yo], axis=2)
            y_ref[s2] = y.reshape(BB, C8, 8, TB).transpose(1, 0, 2, 3)

    def body(x_ref, y_ref, *_):
        compute(x_ref, y_ref)

    in_specs = [pl.BlockSpec((SB * BB, TB // 2, 2 * D), lambda s: (s, 0, 0))]
    args = [x]
    kwargs = {}
    if acc is not None:
        def body(x_ref, acc_ref, y_ref):
            compute(x_ref, y_ref)

        in_specs = in_specs + [pl.BlockSpec(memory_space=pl.ANY)]
        args = [x, acc]
        kwargs = {"input_output_aliases": {1: 0}}

    return pl.pallas_call(
        body,
        grid=(Ssl // SB,),
        in_specs=in_specs,
        out_specs=pl.BlockSpec(
            (SB, C8, BB, 8, TB),
            lambda s: (s + k * (Ssl // SB), 0, 0, 0, 0),
        ),
        out_shape=jax.ShapeDtypeStruct((S, C8, BB, 8, TB), jnp.float32),
        **kwargs,
    )(*args)


def _make_sc_gather(n_units, V2, D):
    upw = n_units // NW
    mesh = plsc.VectorSubcoreMesh(core_axis_name="c", subcore_axis_name="s")
    N = n_units * TB

    @functools.partial(
        pl.kernel,
        mesh=mesh,
        out_type=jax.ShapeDtypeStruct((N, D), jnp.float32),
        scratch_types=[
            pltpu.VMEM((upw, TB), jnp.int32),
            pltpu.VMEM((NBUF, TB, D), jnp.float32),
        ]
        + [pltpu.SemaphoreType.DMA] * (2 * NBUF),
        compiler_params=pltpu.CompilerParams(use_tc_tiling_on_sc=False),
    )
    def sc_gather(idx_hbm, table_hbm, out_hbm, idx_v, rows_v, *sems):
        gsems = sems[:NBUF]
        wsems = sems[NBUF:]
        wid = lax.axis_index("s") * NC + lax.axis_index("c")
        ubase = wid * upw

        pltpu.sync_copy(idx_hbm.at[pl.ds(ubase, upw)], idx_v)

        def gather_start(ul, b):
            pltpu.make_async_copy(
                table_hbm.at[idx_v.at[ul]], rows_v.at[b], gsems[b]
            ).start()

        def gather_wait(b):
            pltpu.make_async_copy(
                table_hbm.at[idx_v.at[0]], rows_v.at[b], gsems[b]
            ).wait()

        def write_start(ul, b):
            pltpu.make_async_copy(
                rows_v.at[b],
                out_hbm.at[pl.ds((ubase + ul) * TB, TB)],
                wsems[b],
            ).start()

        def write_wait(b):
            pltpu.make_async_copy(
                rows_v.at[b], out_hbm.at[pl.ds(0, TB)], wsems[b]
            ).wait()

        for b in range(NBUF):
            gather_start(b, b)

        def outer(g, carry):
            for b in range(NBUF):
                ul = g * NBUF + b
                gather_wait(b)
                write_start(ul, b)
            for b in range(NBUF):
                jn = (g + 1) * NBUF + b

                @pl.when(jn < upw)
                def _():
                    write_wait(b)
                    gather_start(jn, b)

            return carry

        lax.fori_loop(0, upw // NBUF, outer, 0)

        for b in range(NBUF):
            write_wait(b)

    return sc_gather


SLABS = 5


def kernel(sequence, table):
    B, S = sequence.shape
    V, D = table.shape
    BB = B // TB
    n_units = S * BB
    assert n_units % NW == 0 and D % 16 == 0

    table2 = _table_prep(table.T)
    V2 = 2 * table2.shape[0]
    table2v = table2.reshape(V2, D)

    idx = (
        _remap(sequence.T)
        .reshape(n_units, 2, TB // 2)
        .transpose(0, 2, 1)
        .reshape(n_units, TB)
    )
    Ssl = S // SLABS
    usl = Ssl * BB
    sc = _make_sc_gather(usl, V2, D)
    out5d = None
    for k in range(SLABS):
        rows_k = sc(idx[k * usl : (k + 1) * usl], table2v)
        out5d = _out_transpose(
            rows_k.reshape(usl, TB // 2, 2 * D), S, BB, D, Ssl, k, out5d
        )

    return out5d.transpose(2, 4, 0, 1, 3).reshape(B, S, D)
